# baseline (speedup 1.0000x reference)
import functools

import jax
import jax.numpy as jnp
from jax import lax
from jax.experimental import pallas as pl
from jax.experimental.pallas import tpu as pltpu
from jax.experimental.pallas import tpu_sc as plsc

NNODE = 10000
NEDGE = 320000
D = 128
H = 4
DH = 32
LANES = 16
NC = 2
NS = 16
NW = NC * NS
NPAD = 10240
DROWS = 640
TROWS = NPAD + DROWS
TRPT = TROWS // NS
EPAD = NW * 10240
EPW = EPAD // NW
CH = 32
NCHUNK = EPW // CH
GCH = EPAD // CH
NGRP = CH // LANES
PADROW = NNODE + 100
DUMROW = TROWS - 1


def _edge_body(ei_hbm, q_hbm, k_hbm, v_hbm, z_hbm, tab_hbm,
               idx0, idx1, rsc0, rsc1, didx0, didx1,
               qb0, kb0, vb0, qb1, kb1, vb1, norm0, norm1, tab_sh,
               semg0, semg1, semn0, semn1, semd0, semd1):
    c = lax.axis_index("c")
    s = lax.axis_index("s")
    wid = s * NC + c

    idxb = (idx0, idx1)
    rsc = (rsc0, rsc1)
    didx = (didx0, didx1)
    normb = (norm0, norm1)
    semd = (semd0, semd1)
    qb = (qb0, qb1)
    kb = (kb0, kb1)
    vb = (vb0, vb1)
    semg = (semg0, semg1)
    semn = (semn0, semn1)

    r0 = s * TRPT
    pltpu.sync_copy(z_hbm.at[pl.ds(r0, TRPT)], tab_sh.at[pl.ds(r0, TRPT)])
    pltpu.sync_copy(z_hbm.at[pl.ds(0, CH)], norm0)
    pltpu.sync_copy(z_hbm.at[pl.ds(0, CH)], norm1)
    plsc.subcore_barrier()

    iota16 = lax.iota(jnp.int32, LANES)
    wbase = wid * NCHUNK

    def issue_gathers(p):
        rows_ix = idxb[p].at[pl.ds(0, CH)]
        cols_ix = idxb[p].at[pl.ds(CH, CH)]
        pltpu.async_copy(q_hbm.at[rows_ix], qb[p], semg[p])
        pltpu.async_copy(k_hbm.at[cols_ix], kb[p], semg[p])
        pltpu.async_copy(v_hbm.at[cols_ix], vb[p], semg[p])

    def wait_gathers(p):
        rows_ix = idxb[p].at[pl.ds(0, CH)]
        cols_ix = idxb[p].at[pl.ds(CH, CH)]
        pltpu.make_async_copy(q_hbm.at[rows_ix], qb[p], semg[p]).wait()
        pltpu.make_async_copy(k_hbm.at[cols_ix], kb[p], semg[p]).wait()
        pltpu.make_async_copy(v_hbm.at[cols_ix], vb[p], semg[p]).wait()

    def wait_num(p):
        pltpu.make_async_copy(vb[p], tab_sh.at[rsc[p]], semn[p]).wait()

    def compute(p):
        def group(t, _):
            e_vec = t * LANES + iota16
            rv = idxb[p][pl.ds(t * LANES, LANES)]
            rsc[p][pl.ds(t * LANES, LANES)] = rv
            didx[p][pl.ds(t * LANES, LANES)] = (
                lax.shift_right_logical(rv, 4) + NPAD)
            pcol = (rv & (LANES - 1)) * 8
            for h in range(H):
                def dot_step(j, acc):
                    d_vec = h * DH + ((j + iota16) & (DH - 1))
                    qc = plsc.load_gather(qb[p], [e_vec, d_vec])
                    kc = plsc.load_gather(kb[p], [e_vec, d_vec])
                    return acc + qc * kc

                acc = lax.fori_loop(0, DH, dot_step,
                                    jnp.zeros((LANES,), jnp.float32),
                                    unroll=8)
                att = jnp.exp(jnp.clip(acc, -10.0, 10.0))
                plsc.store_scatter(normb[p], [e_vec, pcol + h], att)

                def v_step(j, _):
                    d_vec = h * DH + ((j + iota16) & (DH - 1))
                    vc = plsc.load_gather(vb[p], [e_vec, d_vec])
                    plsc.store_scatter(vb[p], [e_vec, d_vec], vc * att)
                    return 0

                lax.fori_loop(0, DH, v_step, 0, unroll=8)
            return 0

        lax.fori_loop(0, NGRP, group, 0)

    def clear_norm(p):
        def group(t, _):
            e_vec = t * LANES + iota16
            rv = rsc[p][pl.ds(t * LANES, LANES)]
            pcol = (rv & (LANES - 1)) * 8
            zero = jnp.zeros((LANES,), jnp.float32)
            for h in range(H):
                plsc.store_scatter(normb[p], [e_vec, pcol + h], zero)
            return 0

        lax.fori_loop(0, NGRP, group, 0)

    def wait_den(p):
        pltpu.make_async_copy(normb[p], tab_sh.at[didx[p]], semd[p]).wait()

    def body(cix, p, drain, wait_prev_num, prefetch):
        q = 1 - p
        wait_gathers(p)
        if drain:
            wait_den(p)
            clear_norm(p)
        compute(p)
        pltpu.async_copy(vb[p], tab_sh.at[rsc[p]], semn[p], add=True)
        pltpu.async_copy(normb[p], tab_sh.at[didx[p]], semd[p], add=True)
        if prefetch:
            pltpu.sync_copy(ei_hbm.at[cix + 1], idxb[q])
            if wait_prev_num:
                wait_num(q)
            issue_gathers(q)

    pltpu.sync_copy(ei_hbm.at[wbase], idx0)
    issue_gathers(0)
    body(wbase + 0, 0, False, False, True)
    body(wbase + 1, 1, False, True, True)

    def outer(o, carry):
        body(wbase + 2 * o + 2, 0, True, True, True)
        body(wbase + 2 * o + 3, 1, True, True, True)
        return 0

    lax.fori_loop(0, NCHUNK // 2 - 2, outer, 0)
    body(wbase + NCHUNK - 2, 0, True, True, True)
    body(wbase + NCHUNK - 1, 1, True, True, False)
    wait_num(0)
    wait_num(1)
    wait_den(0)
    wait_den(1)

    plsc.subcore_barrier()
    pltpu.sync_copy(tab_sh.at[pl.ds(r0, TRPT)], tab_hbm.at[c, pl.ds(r0, TRPT)])


_edge_pass = functools.partial(
    pl.kernel,
    out_type=jax.ShapeDtypeStruct((NC, TROWS, D), jnp.float32),
    mesh=plsc.VectorSubcoreMesh(core_axis_name="c", subcore_axis_name="s",
                                num_cores=NC, num_subcores=NS),
    compiler_params=pltpu.CompilerParams(needs_layout_passes=False),
    scratch_types=[
        pltpu.VMEM((D,), jnp.int32),
        pltpu.VMEM((D,), jnp.int32),
        pltpu.VMEM((CH,), jnp.int32),
        pltpu.VMEM((CH,), jnp.int32),
        pltpu.VMEM((CH,), jnp.int32),
        pltpu.VMEM((CH,), jnp.int32),
        pltpu.VMEM((CH, D), jnp.float32),
        pltpu.VMEM((CH, D), jnp.float32),
        pltpu.VMEM((CH, D), jnp.float32),
        pltpu.VMEM((CH, D), jnp.float32),
        pltpu.VMEM((CH, D), jnp.float32),
        pltpu.VMEM((CH, D), jnp.float32),
        pltpu.VMEM((CH, D), jnp.float32),
        pltpu.VMEM((CH, D), jnp.float32),
        pltpu.VMEM_SHARED((TROWS, D), jnp.float32),
        pltpu.SemaphoreType.DMA,
        pltpu.SemaphoreType.DMA,
        pltpu.SemaphoreType.DMA,
        pltpu.SemaphoreType.DMA,
        pltpu.SemaphoreType.DMA,
        pltpu.SemaphoreType.DMA,
    ],
)(_edge_body)


def _head_body(x_ref, wp_ref, bp_ref, pos_ref, wqkv_ref,
               emb_ref, q_ref, k_ref, v_ref):
    z = jnp.dot(x_ref[...], wp_ref[...], preferred_element_type=jnp.float32)
    z = z + bp_ref[...] + pos_ref[...]
    emb_ref[...] = z
    qkv = jnp.dot(z, wqkv_ref[...], preferred_element_type=jnp.float32)
    q_ref[...] = qkv[:, :D]
    k_ref[...] = qkv[:, D:2 * D]
    v_ref[...] = qkv[:, 2 * D:]


def _agg_ln(num_ref, den_ref, emb_ref, g_ref, b_ref):
    num = num_ref[0] + num_ref[1]
    den4 = den_ref[...]
    nrow = num.shape[0]
    den128 = jnp.concatenate(
        [jnp.broadcast_to(den4[:, h:h + 1], (nrow, DH)) for h in range(H)],
        axis=1)
    res = num / (den128 + 1e-8) + emb_ref[...]
    mean = jnp.mean(res, axis=-1, keepdims=True)
    cen = res - mean
    var = jnp.mean(cen * cen, axis=-1, keepdims=True)
    return cen * lax.rsqrt(var + 1e-6) * g_ref[...] + b_ref[...]


def _mid_body(num_ref, den_ref, emb_ref, g_ref, b_ref, wqkv_ref,
              y_ref, q_ref, k_ref, v_ref):
    y = _agg_ln(num_ref, den_ref, emb_ref, g_ref, b_ref)
    y_ref[...] = y
    qkv = jnp.dot(y, wqkv_ref[...], preferred_element_type=jnp.float32)
    q_ref[...] = qkv[:, :D]
    k_ref[...] = qkv[:, D:2 * D]
    v_ref[...] = qkv[:, 2 * D:]


def _tail_body(num_ref, den_ref, emb_ref, g_ref, b_ref, w_ref, bias_ref,
               out_ref):
    y = _agg_ln(num_ref, den_ref, emb_ref, g_ref, b_ref)
    out_ref[...] = (
        jnp.dot(y, w_ref[...], preferred_element_type=jnp.float32)
        + bias_ref[...])


_f32 = jnp.float32
BLK = 2000
GRID = NNODE // BLK

_node_spec = pl.BlockSpec((BLK, D), lambda i: (i, 0))
_w128_spec = pl.BlockSpec((D, D), lambda i: (0, 0))
_wqkv_spec = pl.BlockSpec((D, 3 * D), lambda i: (0, 0))
_row_spec = pl.BlockSpec((1, D), lambda i: (0, 0))
_num_spec = pl.BlockSpec((NC, BLK, D), lambda i: (0, i, 0))
_den_spec = pl.BlockSpec((BLK, 8), lambda i: (i, 0))

_head_call = pl.pallas_call(
    _head_body,
    grid=(GRID,),
    in_specs=[_node_spec, _w128_spec, _row_spec, _row_spec, _wqkv_spec],
    out_specs=(_node_spec,) * 4,
    out_shape=(jax.ShapeDtypeStruct((NNODE, D), _f32),) * 4,
)

_mid_call = pl.pallas_call(
    _mid_body,
    grid=(GRID,),
    in_specs=[_num_spec, _den_spec, _node_spec, _row_spec, _row_spec,
              _wqkv_spec],
    out_specs=(_node_spec,) * 4,
    out_shape=(jax.ShapeDtypeStruct((NNODE, D), _f32),) * 4,
)

_tail_call = pl.pallas_call(
    _tail_body,
    grid=(GRID,),
    in_specs=[_num_spec, _den_spec, _node_spec, _row_spec, _row_spec,
              _w128_spec, _row_spec],
    out_specs=_node_spec,
    out_shape=jax.ShapeDtypeStruct((NNODE, D), _f32),
)


def _split_table(tab):
    num = tab[:, :NNODE]
    nd = NNODE // LANES
    denr = tab[0, NPAD:NPAD + nd] + tab[1, NPAD:NPAD + nd]
    den = denr.reshape(NNODE, 8)
    return num, den


@jax.jit
def kernel(graph_node, edge_index, W_P_w, W_P_b, W_pos, qTrans, kTrans,
           vTrans, ln_gamma, ln_beta, invW_w, invW_b):
    rows = edge_index[0].astype(jnp.int32)
    cols = edge_index[1].astype(jnp.int32)
    npad = EPAD - NEDGE
    rows_p = jnp.concatenate([rows, jnp.full((npad,), PADROW, jnp.int32)])
    cols_p = jnp.concatenate([cols, jnp.zeros((npad,), jnp.int32)])
    ei = jnp.concatenate(
        [rows_p.reshape(GCH, CH), cols_p.reshape(GCH, CH),
         jnp.zeros((GCH, D - 2 * CH), jnp.int32)], axis=1)
    wqkv0 = jnp.concatenate([qTrans[0], kTrans[0], vTrans[0]], axis=1)
    wqkv1 = jnp.concatenate([qTrans[1], kTrans[1], vTrans[1]], axis=1)
    zeros = jnp.zeros((TROWS, D), _f32)

    emb0, q0, k0, v0 = _head_call(graph_node, W_P_w, W_P_b.reshape(1, D),
                                  W_pos, wqkv0)
    tab0 = _edge_pass(ei, q0, k0, v0, zeros)
    num0, den0 = _split_table(tab0)
    emb1, q1, k1, v1 = _mid_call(num0, den0, emb0, ln_gamma[0:1],
                                 ln_beta[0:1], wqkv1)
    tab1 = _edge_pass(ei, q1, k1, v1, zeros)
    num1, den1 = _split_table(tab1)
    ret = _tail_call(num1, den1, emb1, ln_gamma[1:2], ln_beta[1:2],
                     invW_w, invW_b.reshape(1, D))
    return ret


# speedup vs baseline: 1.0149x; 1.0149x over previous
import functools

import jax
import jax.numpy as jnp
from jax import lax
from jax.experimental import pallas as pl
from jax.experimental.pallas import tpu as pltpu
from jax.experimental.pallas import tpu_sc as plsc

NNODE = 10000
NEDGE = 320000
D = 128
H = 4
DH = 32
LANES = 16
NC = 2
NS = 16
NW = NC * NS
NPAD = 10240
DROWS = 640
TROWS = NPAD + DROWS
TRPT = TROWS // NS
EPAD = NW * 10240
EPW = EPAD // NW
CH = 32
NCHUNK = EPW // CH
GCH = EPAD // CH
NGRP = CH // LANES
PADROW = NNODE + 100
DUMROW = TROWS - 1


def _edge_body(ei_hbm, q_hbm, kv_hbm, z_hbm, tab_hbm,
               idx0, idx1, rsc0, rsc1, didx,
               qb0, kvb0, ob0, qb1, kvb1, ob1, normbuf, tab_sh,
               semg0, semg1, semn0, semn1, semd):
    c = lax.axis_index("c")
    s = lax.axis_index("s")
    wid = s * NC + c

    idxb = (idx0, idx1)
    rsc = (rsc0, rsc1)
    qb = (qb0, qb1)
    kvb = (kvb0, kvb1)
    ob = (ob0, ob1)
    semg = (semg0, semg1)
    semn = (semn0, semn1)

    r0 = s * TRPT
    pltpu.sync_copy(z_hbm.at[pl.ds(r0, TRPT)], tab_sh.at[pl.ds(r0, TRPT)])
    pltpu.sync_copy(z_hbm.at[pl.ds(0, CH)], normbuf)
    plsc.subcore_barrier()

    iota16 = lax.iota(jnp.int32, LANES)
    wbase = wid * NCHUNK

    def issue_gathers(p):
        rows_ix = idxb[p].at[pl.ds(0, CH)]
        cols_ix = idxb[p].at[pl.ds(CH, CH)]
        pltpu.async_copy(q_hbm.at[rows_ix], qb[p], semg[p])
        pltpu.async_copy(kv_hbm.at[cols_ix], kvb[p], semg[p])

    def wait_gathers(p):
        rows_ix = idxb[p].at[pl.ds(0, CH)]
        cols_ix = idxb[p].at[pl.ds(CH, CH)]
        pltpu.make_async_copy(q_hbm.at[rows_ix], qb[p], semg[p]).wait()
        pltpu.make_async_copy(kv_hbm.at[cols_ix], kvb[p], semg[p]).wait()

    def wait_num(p):
        pltpu.make_async_copy(ob[p], tab_sh.at[rsc[p]], semn[p]).wait()

    def compute(p):
        def group(t, _):
            e_vec = t * LANES + iota16
            rv = idxb[p][pl.ds(t * LANES, LANES)]
            rsc[p][pl.ds(t * LANES, LANES)] = rv
            didx[pl.ds(t * LANES, LANES)] = (
                lax.shift_right_logical(rv, 4) + NPAD)
            pcol = (rv & (LANES - 1)) * 8
            for h in range(H):
                def dot_step(j, acc):
                    d_vec = h * DH + ((j + iota16) & (DH - 1))
                    qc = plsc.load_gather(qb[p], [e_vec, d_vec])
                    kc = plsc.load_gather(kvb[p], [e_vec, d_vec])
                    return acc + qc * kc

                acc = lax.fori_loop(0, DH, dot_step,
                                    jnp.zeros((LANES,), jnp.float32),
                                    unroll=8)
                att = jnp.exp(jnp.clip(acc, -10.0, 10.0))
                plsc.store_scatter(normbuf, [e_vec, pcol + h], att)

                def v_step(j, _):
                    d_vec = h * DH + ((j + iota16) & (DH - 1))
                    vc = plsc.load_gather(kvb[p], [e_vec, D + d_vec])
                    plsc.store_scatter(ob[p], [e_vec, d_vec], vc * att)
                    return 0

                lax.fori_loop(0, DH, v_step, 0, unroll=8)
            return 0

        lax.fori_loop(0, NGRP, group, 0)

    def clear_norm(p):
        def group(t, _):
            e_vec = t * LANES + iota16
            rv = idxb[p][pl.ds(t * LANES, LANES)]
            pcol = (rv & (LANES - 1)) * 8
            zero = jnp.zeros((LANES,), jnp.float32)
            for h in range(H):
                plsc.store_scatter(normbuf, [e_vec, pcol + h], zero)
            return 0

        lax.fori_loop(0, NGRP, group, 0)

    def body(cix, p, prefetch):
        q = 1 - p
        wait_gathers(p)
        compute(p)
        pltpu.async_copy(ob[p], tab_sh.at[rsc[p]], semn[p], add=True)
        pltpu.async_copy(normbuf, tab_sh.at[didx], semd, add=True)
        if prefetch:
            pltpu.sync_copy(ei_hbm.at[cix + 1], idxb[q])
            issue_gathers(q)
        wait_num(p)
        pltpu.make_async_copy(normbuf, tab_sh.at[didx], semd).wait()
        clear_norm(p)

    pltpu.sync_copy(ei_hbm.at[wbase], idx0)
    issue_gathers(0)

    def outer(o, carry):
        body(wbase + 2 * o, 0, True)
        body(wbase + 2 * o + 1, 1, True)
        return 0

    lax.fori_loop(0, NCHUNK // 2 - 1, outer, 0)
    body(wbase + NCHUNK - 2, 0, True)
    body(wbase + NCHUNK - 1, 1, False)

    plsc.subcore_barrier()
    pltpu.sync_copy(tab_sh.at[pl.ds(r0, TRPT)], tab_hbm.at[c, pl.ds(r0, TRPT)])


_edge_pass = functools.partial(
    pl.kernel,
    out_type=jax.ShapeDtypeStruct((NC, TROWS, D), jnp.float32),
    mesh=plsc.VectorSubcoreMesh(core_axis_name="c", subcore_axis_name="s",
                                num_cores=NC, num_subcores=NS),
    compiler_params=pltpu.CompilerParams(needs_layout_passes=False),
    scratch_types=[
        pltpu.VMEM((D,), jnp.int32),
        pltpu.VMEM((D,), jnp.int32),
        pltpu.VMEM((CH,), jnp.int32),
        pltpu.VMEM((CH,), jnp.int32),
        pltpu.VMEM((CH,), jnp.int32),
        pltpu.VMEM((CH, D), jnp.float32),
        pltpu.VMEM((CH, 2 * D), jnp.float32),
        pltpu.VMEM((CH, D), jnp.float32),
        pltpu.VMEM((CH, D), jnp.float32),
        pltpu.VMEM((CH, 2 * D), jnp.float32),
        pltpu.VMEM((CH, D), jnp.float32),
        pltpu.VMEM((CH, D), jnp.float32),
        pltpu.VMEM_SHARED((TROWS, D), jnp.float32),
        pltpu.SemaphoreType.DMA,
        pltpu.SemaphoreType.DMA,
        pltpu.SemaphoreType.DMA,
        pltpu.SemaphoreType.DMA,
        pltpu.SemaphoreType.DMA,
    ],
)(_edge_body)


def _head_body(x_ref, wp_ref, bp_ref, pos_ref, wqkv_ref,
               emb_ref, q_ref, kv_ref):
    z = jnp.dot(x_ref[...], wp_ref[...], preferred_element_type=jnp.float32)
    z = z + bp_ref[...] + pos_ref[...]
    emb_ref[...] = z
    qkv = jnp.dot(z, wqkv_ref[...], preferred_element_type=jnp.float32)
    q_ref[...] = qkv[:, :D]
    kv_ref[...] = qkv[:, D:]


def _agg_ln(num_ref, den_ref, emb_ref, g_ref, b_ref):
    num = num_ref[0] + num_ref[1]
    den4 = den_ref[...]
    nrow = num.shape[0]
    den128 = jnp.concatenate(
        [jnp.broadcast_to(den4[:, h:h + 1], (nrow, DH)) for h in range(H)],
        axis=1)
    res = num / (den128 + 1e-8) + emb_ref[...]
    mean = jnp.mean(res, axis=-1, keepdims=True)
    cen = res - mean
    var = jnp.mean(cen * cen, axis=-1, keepdims=True)
    return cen * lax.rsqrt(var + 1e-6) * g_ref[...] + b_ref[...]


def _mid_body(num_ref, den_ref, emb_ref, g_ref, b_ref, wqkv_ref,
              y_ref, q_ref, kv_ref):
    y = _agg_ln(num_ref, den_ref, emb_ref, g_ref, b_ref)
    y_ref[...] = y
    qkv = jnp.dot(y, wqkv_ref[...], preferred_element_type=jnp.float32)
    q_ref[...] = qkv[:, :D]
    kv_ref[...] = qkv[:, D:]


def _tail_body(num_ref, den_ref, emb_ref, g_ref, b_ref, w_ref, bias_ref,
               out_ref):
    y = _agg_ln(num_ref, den_ref, emb_ref, g_ref, b_ref)
    out_ref[...] = (
        jnp.dot(y, w_ref[...], preferred_element_type=jnp.float32)
        + bias_ref[...])


_f32 = jnp.float32
BLK = 2000
GRID = NNODE // BLK

_node_spec = pl.BlockSpec((BLK, D), lambda i: (i, 0))
_w128_spec = pl.BlockSpec((D, D), lambda i: (0, 0))
_wqkv_spec = pl.BlockSpec((D, 3 * D), lambda i: (0, 0))
_row_spec = pl.BlockSpec((1, D), lambda i: (0, 0))
_num_spec = pl.BlockSpec((NC, BLK, D), lambda i: (0, i, 0))
_kv_spec = pl.BlockSpec((BLK, 2 * D), lambda i: (i, 0))
_den_spec = pl.BlockSpec((BLK, 8), lambda i: (i, 0))

_head_call = pl.pallas_call(
    _head_body,
    grid=(GRID,),
    in_specs=[_node_spec, _w128_spec, _row_spec, _row_spec, _wqkv_spec],
    out_specs=(_node_spec, _node_spec, _kv_spec),
    out_shape=(jax.ShapeDtypeStruct((NNODE, D), _f32),
               jax.ShapeDtypeStruct((NNODE, D), _f32),
               jax.ShapeDtypeStruct((NNODE, 2 * D), _f32)),
)

_mid_call = pl.pallas_call(
    _mid_body,
    grid=(GRID,),
    in_specs=[_num_spec, _den_spec, _node_spec, _row_spec, _row_spec,
              _wqkv_spec],
    out_specs=(_node_spec, _node_spec, _kv_spec),
    out_shape=(jax.ShapeDtypeStruct((NNODE, D), _f32),
               jax.ShapeDtypeStruct((NNODE, D), _f32),
               jax.ShapeDtypeStruct((NNODE, 2 * D), _f32)),
)

_tail_call = pl.pallas_call(
    _tail_body,
    grid=(GRID,),
    in_specs=[_num_spec, _den_spec, _node_spec, _row_spec, _row_spec,
              _w128_spec, _row_spec],
    out_specs=_node_spec,
    out_shape=jax.ShapeDtypeStruct((NNODE, D), _f32),
)


def _split_table(tab):
    num = tab[:, :NNODE]
    nd = NNODE // LANES
    denr = tab[0, NPAD:NPAD + nd] + tab[1, NPAD:NPAD + nd]
    den = denr.reshape(NNODE, 8)
    return num, den


@jax.jit
def kernel(graph_node, edge_index, W_P_w, W_P_b, W_pos, qTrans, kTrans,
           vTrans, ln_gamma, ln_beta, invW_w, invW_b):
    rows = edge_index[0].astype(jnp.int32)
    cols = edge_index[1].astype(jnp.int32)
    npad = EPAD - NEDGE
    rows_p = jnp.concatenate([rows, jnp.full((npad,), PADROW, jnp.int32)])
    cols_p = jnp.concatenate([cols, jnp.zeros((npad,), jnp.int32)])
    ei = jnp.concatenate(
        [rows_p.reshape(GCH, CH), cols_p.reshape(GCH, CH),
         jnp.zeros((GCH, D - 2 * CH), jnp.int32)], axis=1)
    wqkv0 = jnp.concatenate([qTrans[0], kTrans[0], vTrans[0]], axis=1)
    wqkv1 = jnp.concatenate([qTrans[1], kTrans[1], vTrans[1]], axis=1)
    zeros = jnp.zeros((TROWS, D), _f32)

    emb0, q0, kv0 = _head_call(graph_node, W_P_w, W_P_b.reshape(1, D),
                               W_pos, wqkv0)
    tab0 = _edge_pass(ei, q0, kv0, zeros)
    num0, den0 = _split_table(tab0)
    emb1, q1, kv1 = _mid_call(num0, den0, emb0, ln_gamma[0:1],
                              ln_beta[0:1], wqkv1)
    tab1 = _edge_pass(ei, q1, kv1, zeros)
    num1, den1 = _split_table(tab1)
    ret = _tail_call(num1, den1, emb1, ln_gamma[1:2], ln_beta[1:2],
                     invW_w, invW_b.reshape(1, D))
    return ret


# speedup vs baseline: 1.0466x; 1.0312x over previous
import functools

import jax
import jax.numpy as jnp
from jax import lax
from jax.experimental import pallas as pl
from jax.experimental.pallas import tpu as pltpu
from jax.experimental.pallas import tpu_sc as plsc

NNODE = 10000
NEDGE = 320000
D = 128
H = 4
DH = 32
LANES = 16
NC = 2
NS = 16
NW = NC * NS
NPAD = 10240
DROWS = 640
TROWS = NPAD + DROWS
TRPT = TROWS // NS
EPAD = NW * 10240
EPW = EPAD // NW
CH = 32
NCHUNK = EPW // CH
GCH = EPAD // CH
NGRP = CH // LANES
PADROW = NNODE + 100
DUMROW = TROWS - 1


def _edge_body(ei_hbm, q_hbm, kv_hbm, z_hbm, tab_hbm,
               idx0, idx1, rsc0, rsc1, didx,
               qb0, kvb0, ob0, qb1, kvb1, ob1, normbuf, tab_sh,
               semg0, semg1, semn0, semn1, semd):
    c = lax.axis_index("c")
    s = lax.axis_index("s")
    wid = s * NC + c

    idxb = (idx0, idx1)
    rsc = (rsc0, rsc1)
    qb = (qb0, qb1)
    kvb = (kvb0, kvb1)
    ob = (ob0, ob1)
    semg = (semg0, semg1)
    semn = (semn0, semn1)

    r0 = s * TRPT
    pltpu.sync_copy(z_hbm.at[pl.ds(r0, TRPT)], tab_sh.at[pl.ds(r0, TRPT)])
    pltpu.sync_copy(z_hbm.at[pl.ds(0, CH)], normbuf)
    plsc.subcore_barrier()

    iota16 = lax.iota(jnp.int32, LANES)
    wbase = wid * NCHUNK

    def issue_gathers(p):
        rows_ix = idxb[p].at[pl.ds(0, CH)]
        cols_ix = idxb[p].at[pl.ds(CH, CH)]
        pltpu.async_copy(q_hbm.at[rows_ix], qb[p], semg[p])
        pltpu.async_copy(kv_hbm.at[cols_ix], kvb[p], semg[p])

    def wait_gathers(p):
        rows_ix = idxb[p].at[pl.ds(0, CH)]
        cols_ix = idxb[p].at[pl.ds(CH, CH)]
        pltpu.make_async_copy(q_hbm.at[rows_ix], qb[p], semg[p]).wait()
        pltpu.make_async_copy(kv_hbm.at[cols_ix], kvb[p], semg[p]).wait()

    def wait_num(p):
        pltpu.make_async_copy(ob[p], tab_sh.at[rsc[p]], semn[p]).wait()

    def compute(p):
        def group(t, _):
            e_vec = t * LANES + iota16
            rv = idxb[p][pl.ds(t * LANES, LANES)]
            rsc[p][pl.ds(t * LANES, LANES)] = rv
            didx[pl.ds(t * LANES, LANES)] = (
                lax.shift_right_logical(rv, 4) + NPAD)
            pcol = (rv & (LANES - 1)) * 8
            for h in range(H):
                def dot_step(j, acc):
                    d_vec = h * DH + ((j + iota16) & (DH - 1))
                    qc = plsc.load_gather(qb[p], [e_vec, d_vec])
                    kc = plsc.load_gather(kvb[p], [e_vec, d_vec])
                    return acc + qc * kc

                acc = lax.fori_loop(0, DH, dot_step,
                                    jnp.zeros((LANES,), jnp.float32),
                                    unroll=16)
                att = jnp.exp(jnp.clip(acc, -10.0, 10.0))
                plsc.store_scatter(normbuf, [e_vec, pcol + h], att)

                def v_step(j, _):
                    d_vec = h * DH + ((j + iota16) & (DH - 1))
                    vc = plsc.load_gather(kvb[p], [e_vec, D + d_vec])
                    plsc.store_scatter(ob[p], [e_vec, d_vec], vc * att)
                    return 0

                lax.fori_loop(0, DH, v_step, 0, unroll=16)
            return 0

        lax.fori_loop(0, NGRP, group, 0)

    def clear_norm(p):
        def group(t, _):
            e_vec = t * LANES + iota16
            rv = idxb[p][pl.ds(t * LANES, LANES)]
            pcol = (rv & (LANES - 1)) * 8
            zero = jnp.zeros((LANES,), jnp.float32)
            for h in range(H):
                plsc.store_scatter(normbuf, [e_vec, pcol + h], zero)
            return 0

        lax.fori_loop(0, NGRP, group, 0)

    def body(cix, p, prefetch):
        q = 1 - p
        wait_gathers(p)
        compute(p)
        pltpu.async_copy(ob[p], tab_sh.at[rsc[p]], semn[p], add=True)
        pltpu.async_copy(normbuf, tab_sh.at[didx], semd, add=True)
        if prefetch:
            pltpu.sync_copy(ei_hbm.at[cix + 1], idxb[q])
            issue_gathers(q)
        wait_num(p)
        pltpu.make_async_copy(normbuf, tab_sh.at[didx], semd).wait()
        clear_norm(p)

    pltpu.sync_copy(ei_hbm.at[wbase], idx0)
    issue_gathers(0)

    def outer(o, carry):
        body(wbase + 2 * o, 0, True)
        body(wbase + 2 * o + 1, 1, True)
        return 0

    lax.fori_loop(0, NCHUNK // 2 - 1, outer, 0)
    body(wbase + NCHUNK - 2, 0, True)
    body(wbase + NCHUNK - 1, 1, False)

    plsc.subcore_barrier()
    pltpu.sync_copy(tab_sh.at[pl.ds(r0, TRPT)], tab_hbm.at[c, pl.ds(r0, TRPT)])


_edge_pass = functools.partial(
    pl.kernel,
    out_type=jax.ShapeDtypeStruct((NC, TROWS, D), jnp.float32),
    mesh=plsc.VectorSubcoreMesh(core_axis_name="c", subcore_axis_name="s",
                                num_cores=NC, num_subcores=NS),
    compiler_params=pltpu.CompilerParams(needs_layout_passes=False),
    scratch_types=[
        pltpu.VMEM((D,), jnp.int32),
        pltpu.VMEM((D,), jnp.int32),
        pltpu.VMEM((CH,), jnp.int32),
        pltpu.VMEM((CH,), jnp.int32),
        pltpu.VMEM((CH,), jnp.int32),
        pltpu.VMEM((CH, D), jnp.float32),
        pltpu.VMEM((CH, 2 * D), jnp.float32),
        pltpu.VMEM((CH, D), jnp.float32),
        pltpu.VMEM((CH, D), jnp.float32),
        pltpu.VMEM((CH, 2 * D), jnp.float32),
        pltpu.VMEM((CH, D), jnp.float32),
        pltpu.VMEM((CH, D), jnp.float32),
        pltpu.VMEM_SHARED((TROWS, D), jnp.float32),
        pltpu.SemaphoreType.DMA,
        pltpu.SemaphoreType.DMA,
        pltpu.SemaphoreType.DMA,
        pltpu.SemaphoreType.DMA,
        pltpu.SemaphoreType.DMA,
    ],
)(_edge_body)


def _head_body(x_ref, wp_ref, bp_ref, pos_ref, wqkv_ref,
               emb_ref, q_ref, kv_ref):
    z = jnp.dot(x_ref[...], wp_ref[...], preferred_element_type=jnp.float32)
    z = z + bp_ref[...] + pos_ref[...]
    emb_ref[...] = z
    qkv = jnp.dot(z, wqkv_ref[...], preferred_element_type=jnp.float32)
    q_ref[...] = qkv[:, :D]
    kv_ref[...] = qkv[:, D:]


def _agg_ln(num_ref, den_ref, emb_ref, g_ref, b_ref):
    num = num_ref[0] + num_ref[1]
    den4 = den_ref[...]
    nrow = num.shape[0]
    den128 = jnp.concatenate(
        [jnp.broadcast_to(den4[:, h:h + 1], (nrow, DH)) for h in range(H)],
        axis=1)
    res = num / (den128 + 1e-8) + emb_ref[...]
    mean = jnp.mean(res, axis=-1, keepdims=True)
    cen = res - mean
    var = jnp.mean(cen * cen, axis=-1, keepdims=True)
    return cen * lax.rsqrt(var + 1e-6) * g_ref[...] + b_ref[...]


def _mid_body(num_ref, den_ref, emb_ref, g_ref, b_ref, wqkv_ref,
              y_ref, q_ref, kv_ref):
    y = _agg_ln(num_ref, den_ref, emb_ref, g_ref, b_ref)
    y_ref[...] = y
    qkv = jnp.dot(y, wqkv_ref[...], preferred_element_type=jnp.float32)
    q_ref[...] = qkv[:, :D]
    kv_ref[...] = qkv[:, D:]


def _tail_body(num_ref, den_ref, emb_ref, g_ref, b_ref, w_ref, bias_ref,
               out_ref):
    y = _agg_ln(num_ref, den_ref, emb_ref, g_ref, b_ref)
    out_ref[...] = (
        jnp.dot(y, w_ref[...], preferred_element_type=jnp.float32)
        + bias_ref[...])


_f32 = jnp.float32
BLK = 2000
GRID = NNODE // BLK

_node_spec = pl.BlockSpec((BLK, D), lambda i: (i, 0))
_w128_spec = pl.BlockSpec((D, D), lambda i: (0, 0))
_wqkv_spec = pl.BlockSpec((D, 3 * D), lambda i: (0, 0))
_row_spec = pl.BlockSpec((1, D), lambda i: (0, 0))
_num_spec = pl.BlockSpec((NC, BLK, D), lambda i: (0, i, 0))
_kv_spec = pl.BlockSpec((BLK, 2 * D), lambda i: (i, 0))
_den_spec = pl.BlockSpec((BLK, 8), lambda i: (i, 0))

_head_call = pl.pallas_call(
    _head_body,
    grid=(GRID,),
    in_specs=[_node_spec, _w128_spec, _row_spec, _row_spec, _wqkv_spec],
    out_specs=(_node_spec, _node_spec, _kv_spec),
    out_shape=(jax.ShapeDtypeStruct((NNODE, D), _f32),
               jax.ShapeDtypeStruct((NNODE, D), _f32),
               jax.ShapeDtypeStruct((NNODE, 2 * D), _f32)),
)

_mid_call = pl.pallas_call(
    _mid_body,
    grid=(GRID,),
    in_specs=[_num_spec, _den_spec, _node_spec, _row_spec, _row_spec,
              _wqkv_spec],
    out_specs=(_node_spec, _node_spec, _kv_spec),
    out_shape=(jax.ShapeDtypeStruct((NNODE, D), _f32),
               jax.ShapeDtypeStruct((NNODE, D), _f32),
               jax.ShapeDtypeStruct((NNODE, 2 * D), _f32)),
)

_tail_call = pl.pallas_call(
    _tail_body,
    grid=(GRID,),
    in_specs=[_num_spec, _den_spec, _node_spec, _row_spec, _row_spec,
              _w128_spec, _row_spec],
    out_specs=_node_spec,
    out_shape=jax.ShapeDtypeStruct((NNODE, D), _f32),
)


def _split_table(tab):
    num = tab[:, :NNODE]
    nd = NNODE // LANES
    denr = tab[0, NPAD:NPAD + nd] + tab[1, NPAD:NPAD + nd]
    den = denr.reshape(NNODE, 8)
    return num, den


@jax.jit
def kernel(graph_node, edge_index, W_P_w, W_P_b, W_pos, qTrans, kTrans,
           vTrans, ln_gamma, ln_beta, invW_w, invW_b):
    rows = edge_index[0].astype(jnp.int32)
    cols = edge_index[1].astype(jnp.int32)
    npad = EPAD - NEDGE
    rows_p = jnp.concatenate([rows, jnp.full((npad,), PADROW, jnp.int32)])
    cols_p = jnp.concatenate([cols, jnp.zeros((npad,), jnp.int32)])
    ei = jnp.concatenate(
        [rows_p.reshape(GCH, CH), cols_p.reshape(GCH, CH),
         jnp.zeros((GCH, D - 2 * CH), jnp.int32)], axis=1)
    wqkv0 = jnp.concatenate([qTrans[0], kTrans[0], vTrans[0]], axis=1)
    wqkv1 = jnp.concatenate([qTrans[1], kTrans[1], vTrans[1]], axis=1)
    zeros = jnp.zeros((TROWS, D), _f32)

    emb0, q0, kv0 = _head_call(graph_node, W_P_w, W_P_b.reshape(1, D),
                               W_pos, wqkv0)
    tab0 = _edge_pass(ei, q0, kv0, zeros)
    num0, den0 = _split_table(tab0)
    emb1, q1, kv1 = _mid_call(num0, den0, emb0, ln_gamma[0:1],
                              ln_beta[0:1], wqkv1)
    tab1 = _edge_pass(ei, q1, kv1, zeros)
    num1, den1 = _split_table(tab1)
    ret = _tail_call(num1, den1, emb1, ln_gamma[1:2], ln_beta[1:2],
                     invW_w, invW_b.reshape(1, D))
    return ret


# speedup vs baseline: 1.0501x; 1.0033x over previous
import functools

import jax
import jax.numpy as jnp
from jax import lax
from jax.experimental import pallas as pl
from jax.experimental.pallas import tpu as pltpu
from jax.experimental.pallas import tpu_sc as plsc

NNODE = 10000
NEDGE = 320000
D = 128
H = 4
DH = 32
LANES = 16
NC = 2
NS = 16
NW = NC * NS
NPAD = 10240
DROWS = 640
TROWS = NPAD + DROWS
TRPT = TROWS // NS
EPAD = NW * 10240
EPW = EPAD // NW
CH = 32
NCHUNK = EPW // CH
GCH = EPAD // CH
NGRP = CH // LANES
PADROW = NNODE + 100


def _edge_body(ei_hbm, q_hbm, kv_hbm, z_hbm, tab_hbm,
               idx0, idx1, rsc0, rsc1, didx,
               qb0, kvb0, ob0, qb1, kvb1, ob1, normbuf, tab_sh,
               semg0, semg1, semn0, semn1, semd):
    c = lax.axis_index("c")
    s = lax.axis_index("s")
    wid = s * NC + c

    idxb = (idx0, idx1)
    rsc = (rsc0, rsc1)
    qb = (qb0, qb1)
    kvb = (kvb0, kvb1)
    ob = (ob0, ob1)
    semg = (semg0, semg1)
    semn = (semn0, semn1)

    r0 = s * TRPT
    pltpu.sync_copy(z_hbm.at[pl.ds(r0, TRPT)], tab_sh.at[pl.ds(r0, TRPT)])
    pltpu.sync_copy(z_hbm.at[pl.ds(0, CH)], normbuf)
    plsc.subcore_barrier()

    iota16 = lax.iota(jnp.int32, LANES)
    wbase = wid * NCHUNK

    def issue_gathers(p):
        rows_ix = idxb[p].at[pl.ds(0, CH)]
        cols_ix = idxb[p].at[pl.ds(CH, CH)]
        pltpu.async_copy(q_hbm.at[rows_ix], qb[p], semg[p])
        pltpu.async_copy(kv_hbm.at[cols_ix], kvb[p], semg[p])

    def wait_gathers(p):
        rows_ix = idxb[p].at[pl.ds(0, CH)]
        cols_ix = idxb[p].at[pl.ds(CH, CH)]
        pltpu.make_async_copy(q_hbm.at[rows_ix], qb[p], semg[p]).wait()
        pltpu.make_async_copy(kv_hbm.at[cols_ix], kvb[p], semg[p]).wait()

    def wait_num(p):
        pltpu.make_async_copy(ob[p], tab_sh.at[rsc[p]], semn[p]).wait()

    def compute(p):
        def group(t, _):
            e_vec = t * LANES + iota16
            rv = idxb[p][pl.ds(t * LANES, LANES)]
            rsc[p][pl.ds(t * LANES, LANES)] = rv
            didx[pl.ds(t * LANES, LANES)] = (
                lax.shift_right_logical(rv, 4) + NPAD)
            pcol = (rv & (LANES - 1)) * 8
            for h in range(H):
                def dot_step(j, acc):
                    d_vec = h * DH + ((j + iota16) & (DH - 1))
                    qc = plsc.load_gather(qb[p], [e_vec, d_vec])
                    kc = plsc.load_gather(kvb[p], [e_vec, d_vec])
                    return acc + qc * kc

                acc = lax.fori_loop(0, DH, dot_step,
                                    jnp.zeros((LANES,), jnp.float32),
                                    unroll=16)
                att = jnp.exp(jnp.clip(acc, -10.0, 10.0))
                plsc.store_scatter(normbuf, [e_vec, pcol + h], att)

                def v_step(j, _):
                    d_vec = h * DH + ((j + iota16) & (DH - 1))
                    vc = plsc.load_gather(kvb[p], [e_vec, D + d_vec])
                    plsc.store_scatter(ob[p], [e_vec, d_vec], vc * att)
                    return 0

                lax.fori_loop(0, DH, v_step, 0, unroll=16)
            return 0

        lax.fori_loop(0, NGRP, group, 0)

    def clear_norm(p):
        def group(t, _):
            e_vec = t * LANES + iota16
            rv = idxb[p][pl.ds(t * LANES, LANES)]
            pcol = (rv & (LANES - 1)) * 8
            zero = jnp.zeros((LANES,), jnp.float32)
            for h in range(H):
                plsc.store_scatter(normbuf, [e_vec, pcol + h], zero)
            return 0

        lax.fori_loop(0, NGRP, group, 0)

    def body(cix, p, prefetch):
        q = 1 - p
        wait_gathers(p)
        compute(p)
        pltpu.async_copy(ob[p], tab_sh.at[rsc[p]], semn[p], add=True)
        pltpu.async_copy(normbuf, tab_sh.at[didx], semd, add=True)
        if prefetch:
            pltpu.sync_copy(ei_hbm.at[cix + 1], idxb[q])
            issue_gathers(q)
        wait_num(p)
        pltpu.make_async_copy(normbuf, tab_sh.at[didx], semd).wait()
        clear_norm(p)

    pltpu.sync_copy(ei_hbm.at[wbase], idx0)
    issue_gathers(0)

    def outer(o, carry):
        body(wbase + 2 * o, 0, True)
        body(wbase + 2 * o + 1, 1, True)
        return 0

    lax.fori_loop(0, NCHUNK // 2 - 1, outer, 0)
    body(wbase + NCHUNK - 2, 0, True)
    body(wbase + NCHUNK - 1, 1, False)

    plsc.subcore_barrier()
    pltpu.sync_copy(tab_sh.at[pl.ds(r0, TRPT)], tab_hbm.at[c, pl.ds(r0, TRPT)])


_edge_pass = functools.partial(
    pl.kernel,
    out_type=jax.ShapeDtypeStruct((NC, TROWS, D), jnp.float32),
    mesh=plsc.VectorSubcoreMesh(core_axis_name="c", subcore_axis_name="s",
                                num_cores=NC, num_subcores=NS),
    compiler_params=pltpu.CompilerParams(needs_layout_passes=False),
    scratch_types=[
        pltpu.VMEM((D,), jnp.int32),
        pltpu.VMEM((D,), jnp.int32),
        pltpu.VMEM((CH,), jnp.int32),
        pltpu.VMEM((CH,), jnp.int32),
        pltpu.VMEM((CH,), jnp.int32),
        pltpu.VMEM((CH, D), jnp.float32),
        pltpu.VMEM((CH, 2 * D), jnp.float32),
        pltpu.VMEM((CH, D), jnp.float32),
        pltpu.VMEM((CH, D), jnp.float32),
        pltpu.VMEM((CH, 2 * D), jnp.float32),
        pltpu.VMEM((CH, D), jnp.float32),
        pltpu.VMEM((CH, D), jnp.float32),
        pltpu.VMEM_SHARED((TROWS, D), jnp.float32),
        pltpu.SemaphoreType.DMA,
        pltpu.SemaphoreType.DMA,
        pltpu.SemaphoreType.DMA,
        pltpu.SemaphoreType.DMA,
        pltpu.SemaphoreType.DMA,
    ],
)(_edge_body)


def _head_body(x_ref, wp_ref, bp_ref, pos_ref, wqkv_ref,
               emb_ref, q_ref, kv_ref):
    z = jnp.dot(x_ref[...], wp_ref[...], preferred_element_type=jnp.float32)
    z = z + bp_ref[...] + pos_ref[...]
    emb_ref[...] = z
    qkv = jnp.dot(z, wqkv_ref[...], preferred_element_type=jnp.float32)
    q_ref[...] = qkv[:, :D]
    kv_ref[...] = qkv[:, D:]


def _agg_ln(num_ref, den_ref, emb_ref, g_ref, b_ref):
    num = num_ref[0] + num_ref[1]
    den4 = den_ref[...]
    nrow = num.shape[0]
    den128 = jnp.concatenate(
        [jnp.broadcast_to(den4[:, h:h + 1], (nrow, DH)) for h in range(H)],
        axis=1)
    res = num / (den128 + 1e-8) + emb_ref[...]
    mean = jnp.mean(res, axis=-1, keepdims=True)
    cen = res - mean
    var = jnp.mean(cen * cen, axis=-1, keepdims=True)
    return cen * lax.rsqrt(var + 1e-6) * g_ref[...] + b_ref[...]


def _mid_body(num_ref, den_ref, emb_ref, g_ref, b_ref, wqkv_ref,
              y_ref, q_ref, kv_ref):
    y = _agg_ln(num_ref, den_ref, emb_ref, g_ref, b_ref)
    y_ref[...] = y
    qkv = jnp.dot(y, wqkv_ref[...], preferred_element_type=jnp.float32)
    q_ref[...] = qkv[:, :D]
    kv_ref[...] = qkv[:, D:]


def _tail_body(num_ref, den_ref, emb_ref, g_ref, b_ref, w_ref, bias_ref,
               out_ref):
    y = _agg_ln(num_ref, den_ref, emb_ref, g_ref, b_ref)
    out_ref[...] = (
        jnp.dot(y, w_ref[...], preferred_element_type=jnp.float32)
        + bias_ref[...])


_f32 = jnp.float32
BLK = 2000
GRID = NNODE // BLK

_node_spec = pl.BlockSpec((BLK, D), lambda i: (i, 0))
_w128_spec = pl.BlockSpec((D, D), lambda i: (0, 0))
_wqkv_spec = pl.BlockSpec((D, 3 * D), lambda i: (0, 0))
_row_spec = pl.BlockSpec((1, D), lambda i: (0, 0))
_num_spec = pl.BlockSpec((NC, BLK, D), lambda i: (0, i, 0))
_kv_spec = pl.BlockSpec((BLK, 2 * D), lambda i: (i, 0))
_den_spec = pl.BlockSpec((BLK, 8), lambda i: (i, 0))

_head_call = pl.pallas_call(
    _head_body,
    grid=(GRID,),
    in_specs=[_node_spec, _w128_spec, _row_spec, _row_spec, _wqkv_spec],
    out_specs=(_node_spec, _node_spec, _kv_spec),
    out_shape=(jax.ShapeDtypeStruct((NNODE, D), _f32),
               jax.ShapeDtypeStruct((NNODE, D), _f32),
               jax.ShapeDtypeStruct((NNODE, 2 * D), _f32)),
)

_mid_call = pl.pallas_call(
    _mid_body,
    grid=(GRID,),
    in_specs=[_num_spec, _den_spec, _node_spec, _row_spec, _row_spec,
              _wqkv_spec],
    out_specs=(_node_spec, _node_spec, _kv_spec),
    out_shape=(jax.ShapeDtypeStruct((NNODE, D), _f32),
               jax.ShapeDtypeStruct((NNODE, D), _f32),
               jax.ShapeDtypeStruct((NNODE, 2 * D), _f32)),
)

_tail_call = pl.pallas_call(
    _tail_body,
    grid=(GRID,),
    in_specs=[_num_spec, _den_spec, _node_spec, _row_spec, _row_spec,
              _w128_spec, _row_spec],
    out_specs=_node_spec,
    out_shape=jax.ShapeDtypeStruct((NNODE, D), _f32),
)


def _split_table(tab):
    num = tab[:, :NNODE]
    nd = NNODE // LANES
    denr = tab[0, NPAD:NPAD + nd] + tab[1, NPAD:NPAD + nd]
    den = denr.reshape(NNODE, 8)
    return num, den


@jax.jit
def kernel(graph_node, edge_index, W_P_w, W_P_b, W_pos, qTrans, kTrans,
           vTrans, ln_gamma, ln_beta, invW_w, invW_b):
    rows = edge_index[0].astype(jnp.int32)
    cols = edge_index[1].astype(jnp.int32)
    npad = EPAD - NEDGE
    rows_p = jnp.concatenate([rows, jnp.full((npad,), PADROW, jnp.int32)])
    cols_p = jnp.concatenate([cols, jnp.zeros((npad,), jnp.int32)])
    ei = jnp.concatenate(
        [rows_p.reshape(GCH, CH), cols_p.reshape(GCH, CH),
         jnp.zeros((GCH, D - 2 * CH), jnp.int32)], axis=1)
    wqkv0 = jnp.concatenate([qTrans[0], kTrans[0], vTrans[0]], axis=1)
    wqkv1 = jnp.concatenate([qTrans[1], kTrans[1], vTrans[1]], axis=1)
    zeros = jnp.zeros((TROWS, D), _f32)

    emb0, q0, kv0 = _head_call(graph_node, W_P_w, W_P_b.reshape(1, D),
                               W_pos, wqkv0)
    tab0 = _edge_pass(ei, q0, kv0, zeros)
    num0, den0 = _split_table(tab0)
    emb1, q1, kv1 = _mid_call(num0, den0, emb0, ln_gamma[0:1],
                              ln_beta[0:1], wqkv1)
    tab1 = _edge_pass(ei, q1, kv1, zeros)
    num1, den1 = _split_table(tab1)
    ret = _tail_call(num1, den1, emb1, ln_gamma[1:2], ln_beta[1:2],
                     invW_w, invW_b.reshape(1, D))
    return ret
